# baseline (speedup 1.0000x reference)
import jax
import jax.numpy as jnp
from jax.experimental import pallas as pl
from jax.experimental.pallas import tpu as pltpu

_ROWS2D = 37500
_BR = 3840
_GRID = -(-_ROWS2D // _BR)


def _body(x_ref, c_ref, o_ref):
    o_ref[...] = x_ref[...]
    @pl.when(pl.program_id(0) == 0)
    def _():
        o_ref[0:3, :] = o_ref[0:3, :] + c_ref[...]


def kernel(x):
    t = jnp.arange(0, 336, 1, dtype=jnp.float32).reshape(7, 6, 8)
    addflat = jnp.zeros((3 * 128,), jnp.float32)
    addflat = addflat.at[0:336].set(t[jnp.array([0, 2, 1, 3, 4, 5, 6])].reshape(336))
    addflat = addflat.reshape(3, 128)
    xr = x.reshape(_ROWS2D, 128)
    res = pl.pallas_call(
        _body,
        grid=(_GRID,),
        in_specs=[
            pl.BlockSpec((_BR, 128), lambda i: (i, 0)),
            pl.BlockSpec((3, 128), lambda i: (0, 0)),
        ],
        out_specs=pl.BlockSpec((_BR, 128), lambda i: (i, 0)),
        out_shape=jax.ShapeDtypeStruct((_ROWS2D, 128), jnp.float32),
        compiler_params=pltpu.CompilerParams(
            dimension_semantics=("arbitrary",),
        ),
    )(xr, addflat)
    return res.reshape(100000, 6, 8)


# speedup vs baseline: 1.0001x; 1.0001x over previous
import jax
import jax.numpy as jnp
from jax.experimental import pallas as pl
from jax.experimental.pallas import tpu as pltpu

_ROWS2D = 37500
_BR = 37500
_GRID = -(-_ROWS2D // _BR)


def _body(x_ref, c_ref, o_ref):
    o_ref[...] = x_ref[...]
    @pl.when(pl.program_id(0) == 0)
    def _():
        o_ref[0:3, :] = o_ref[0:3, :] + c_ref[...]


def kernel(x):
    t = jnp.arange(0, 336, 1, dtype=jnp.float32).reshape(7, 6, 8)
    addflat = jnp.zeros((3 * 128,), jnp.float32)
    addflat = addflat.at[0:336].set(t[jnp.array([0, 2, 1, 3, 4, 5, 6])].reshape(336))
    addflat = addflat.reshape(3, 128)
    xr = x.reshape(_ROWS2D, 128)
    res = pl.pallas_call(
        _body,
        grid=(_GRID,),
        in_specs=[
            pl.BlockSpec((_BR, 128), lambda i: (i, 0)),
            pl.BlockSpec((3, 128), lambda i: (0, 0)),
        ],
        out_specs=pl.BlockSpec((_BR, 128), lambda i: (i, 0)),
        out_shape=jax.ShapeDtypeStruct((_ROWS2D, 128), jnp.float32),
        compiler_params=pltpu.CompilerParams(
            dimension_semantics=("arbitrary",),
        ),
    )(xr, addflat)
    return res.reshape(100000, 6, 8)


# speedup vs baseline: 1.7366x; 1.7364x over previous
import jax
import jax.numpy as jnp
from jax.experimental import pallas as pl
from jax.experimental.pallas import tpu as pltpu

_N = 100000


def _body(x_ref, c_ref, o_ref):
    o_ref[...] = x_ref[...] + c_ref[...]


def kernel(x):
    t = jnp.arange(0, 336, 1, dtype=jnp.float32).reshape(7, 6, 8)
    addvals = jnp.concatenate(
        [t[jnp.array([0, 2, 1, 3, 4, 5, 6])], jnp.zeros((1, 6, 8), jnp.float32)], axis=0
    )
    return pl.pallas_call(
        _body,
        grid=(1,),
        in_specs=[
            pl.BlockSpec((8, 6, 8), lambda i: (0, 0, 0)),
            pl.BlockSpec((8, 6, 8), lambda i: (0, 0, 0)),
        ],
        out_specs=pl.BlockSpec((8, 6, 8), lambda i: (0, 0, 0)),
        out_shape=jax.ShapeDtypeStruct((_N, 6, 8), jnp.float32),
        input_output_aliases={0: 0},
    )(x, addvals)
